# baseline (speedup 1.0000x reference)
import functools

import jax
import jax.numpy as jnp
from jax import lax
from jax.experimental import pallas as pl
from jax.experimental.pallas import tpu as pltpu
from jax.experimental.pallas import tpu_sc as plsc

NC = 2
NS = 16
NW = NC * NS

BATCH = 16384
D_EMB = 123
D_OUT = 128
B_PER_W = BATCH // NW
N_CHUNK = B_PER_W // 128


def _sc_body(idx_hbm, table_hbm, b1_hbm, w2_hbm, b2_hbm, w3_hbm, b3_hbm,
             out_hbm,
             idx_v, out_v, b1_v, w2_v, b2_v, w3_v, b3_v,
             sem):
    wid = lax.axis_index("s") * NC + lax.axis_index("c")
    base = wid * B_PER_W

    pltpu.sync_copy(idx_hbm.at[wid], idx_v)
    copies = []
    for j in range(N_CHUNK):
        copies.append(
            pltpu.async_copy(table_hbm.at[idx_v.at[j]],
                             out_v.at[pl.ds(j * 128, 128)], sem))

    pltpu.sync_copy(b1_hbm, b1_v)
    pltpu.sync_copy(w2_hbm, w2_v)
    pltpu.sync_copy(b2_hbm, b2_v)
    pltpu.sync_copy(w3_hbm, w3_v)
    pltpu.sync_copy(b3_hbm, b3_v)

    b1a = b1_v[pl.ds(0, 16)]
    b1b = b1_v[pl.ds(16, 16)]
    t0 = b2_v[pl.ds(0, 16)]
    t1 = b2_v[pl.ds(16, 16)]
    for k in range(30):
        bk = b1a[k] if k < 16 else b1b[k - 16]
        t0 = t0 + bk * w2_v[k, pl.ds(0, 16)]
        t1 = t1 + bk * w2_v[k, pl.ds(16, 16)]

    h = b3_v[pl.ds(0, 16)]
    for k in range(30):
        tk = t0[k] if k < 16 else t1[k - 16]
        h = h + tk * w3_v[k, pl.ds(0, 16)]

    for j in range(N_CHUNK):
        copies[j].wait()
        for r in range(j * 128, (j + 1) * 128):
            sl = (r, pl.ds(D_OUT - 16, 16))
            out_v[sl] = out_v[sl] + h

    pltpu.sync_copy(out_v, out_hbm.at[pl.ds(base, B_PER_W)])


@jax.jit
def _sc_call(idx3d, table_pad, b1p, W2p, b2p, W3p, b3p):
    mesh = plsc.VectorSubcoreMesh(core_axis_name="c", subcore_axis_name="s")
    run = functools.partial(
        pl.kernel,
        out_type=jax.ShapeDtypeStruct((BATCH, D_OUT), jnp.float32),
        mesh=mesh,
        scratch_types=[
            pltpu.VMEM((N_CHUNK, 128), jnp.int32),
            pltpu.VMEM((B_PER_W, D_OUT), jnp.float32),
            pltpu.VMEM((32,), jnp.float32),
            pltpu.VMEM((30, 32), jnp.float32),
            pltpu.VMEM((32,), jnp.float32),
            pltpu.VMEM((30, 16), jnp.float32),
            pltpu.VMEM((16,), jnp.float32),
            pltpu.SemaphoreType.DMA,
        ],
    )(_sc_body)
    return run(idx3d, table_pad, b1p, W2p, b2p, W3p, b3p)


def kernel(item_inputs, W_emb, W1, b1, W2, b2, W3, b3):
    del W1
    idx3d = item_inputs.reshape(NW, N_CHUNK, 128)
    table_pad = jnp.pad(W_emb, ((0, 0), (0, D_OUT - D_EMB)))
    b1p = jnp.pad(b1, (0, 2))
    W2p = jnp.pad(W2, ((0, 0), (0, 2)))
    b2p = jnp.pad(b2, (0, 2))
    W3p = jnp.pad(W3, ((0, 0), (11, 0)))
    b3p = jnp.pad(b3, (11, 0))
    return _sc_call(idx3d, table_pad, b1p, W2p, b2p, W3p, b3p)
